# baseline (speedup 1.0000x reference)
import functools

import jax
import jax.numpy as jnp
from jax import lax
from jax.experimental import pallas as pl
from jax.experimental.pallas import tpu as pltpu
from jax.experimental.pallas import tpu_sc as plsc

NC, NS = 2, 16
NW = NC * NS
OUT_D = 10
PAD_D = 17
L = 16

VOCAB = 1000
BATCH, SEQ = 4096, 200
TOTAL = BATCH * SEQ
N_PER_W = TOTAL // NW
CHUNK = 2560
N_CHUNKS = N_PER_W // CHUNK
GROUPS = CHUNK // L


def _fuse_table_body(emb_ref, w_ref, b_ref, out_ref):
    out_ref[...] = (
        jnp.dot(emb_ref[...], w_ref[...], preferred_element_type=jnp.float32)
        + b_ref[...]
    )


def _pad_wb(W, b):
    wp = jnp.zeros((W.shape[0], PAD_D), jnp.float32).at[:, :OUT_D].set(W)
    bp = jnp.zeros((1, PAD_D), jnp.float32).at[0, :OUT_D].set(b)
    return wp, bp


def _gather_body(fused_hbm, idx_hbm, out_hbm, table_v, idx_v, out_v, sem):
    wid = lax.axis_index("s") * NC + lax.axis_index("c")
    base = wid * N_PER_W

    pltpu.sync_copy(fused_hbm, table_v)

    iota = lax.iota(jnp.int32, L)
    col_pos = [iota * OUT_D + c for c in range(OUT_D)]

    def chunk(g, carry):
        off = base + g * CHUNK
        pltpu.sync_copy(idx_hbm.at[pl.ds(off, CHUNK)], idx_v)

        @plsc.parallel_loop(0, GROUPS, unroll=8)
        def group(t):
            rows = idx_v[pl.ds(t * L, L)] * PAD_D
            obase = t * (L * OUT_D)
            for c in range(OUT_D):
                vals = plsc.load_gather(table_v, [rows + c])
                plsc.store_scatter(out_v, [col_pos[c] + obase], vals)
        pltpu.sync_copy(out_v, out_hbm.at[pl.ds(off * OUT_D, CHUNK * OUT_D)])
        return carry

    lax.fori_loop(0, N_CHUNKS, chunk, 0)


EXP_R = 8192


def _expand_body(in_ref, out_ref):
    out_ref[...] = in_ref[...].reshape(EXP_R, OUT_D)


def kernel(batch, emb_table, W, b):
    wp, bp = _pad_wb(W, b)
    fused = pl.pallas_call(
        _fuse_table_body,
        out_shape=jax.ShapeDtypeStruct((VOCAB, PAD_D), jnp.float32),
    )(emb_table, wp, bp).reshape(VOCAB * PAD_D)

    idx = batch.reshape(TOTAL)

    mesh = plsc.VectorSubcoreMesh(core_axis_name="c", subcore_axis_name="s")
    flat = pl.kernel(
        _gather_body,
        out_type=jax.ShapeDtypeStruct((TOTAL * OUT_D,), jnp.float32),
        mesh=mesh,
        scratch_types=[
            pltpu.VMEM((VOCAB * PAD_D,), jnp.float32),
            pltpu.VMEM((CHUNK,), jnp.int32),
            pltpu.VMEM((CHUNK * OUT_D,), jnp.float32),
            pltpu.SemaphoreType.DMA,
        ],
        compiler_params=pltpu.CompilerParams(
            use_tc_tiling_on_sc=False, needs_layout_passes=False
        ),
    )(fused, idx)

    return flat.reshape(BATCH, SEQ, OUT_D) + b[0] * 0.0


# speedup vs baseline: 1.3173x; 1.3173x over previous
import functools

import jax
import jax.numpy as jnp
from jax import lax
from jax.experimental import pallas as pl
from jax.experimental.pallas import tpu as pltpu
from jax.experimental.pallas import tpu_sc as plsc

NC, NS = 2, 16
NW = NC * NS
OUT_D = 10
PAD_D = 17
L = 16

VOCAB = 1000
BATCH, SEQ = 4096, 200
TOTAL = BATCH * SEQ
N_PER_W = TOTAL // NW
CHUNK = 2560
N_CHUNKS = N_PER_W // CHUNK
GROUPS = CHUNK // L


def _fuse_table_body(emb_ref, w_ref, b_ref, out_ref):
    out_ref[...] = (
        jnp.dot(emb_ref[...], w_ref[...], preferred_element_type=jnp.float32)
        + b_ref[...]
    )


def _pad_wb(W, b):
    wp = jnp.zeros((W.shape[0], PAD_D), jnp.float32).at[:, :OUT_D].set(W)
    bp = jnp.zeros((1, PAD_D), jnp.float32).at[0, :OUT_D].set(b)
    return wp, bp


def _gather_body(fused_hbm, idx_hbm, out_hbm, table_v, idx_v, out_v, sem):
    wid = lax.axis_index("s") * NC + lax.axis_index("c")
    base = wid * N_PER_W

    pltpu.sync_copy(fused_hbm, table_v)

    iota = lax.iota(jnp.int32, L)
    col_sel = [jnp.full((L,), c, jnp.int32) for c in range(OUT_D)]

    def chunk(g, carry):
        off = base + g * CHUNK
        pltpu.sync_copy(idx_hbm.at[pl.ds(off, CHUNK)], idx_v)

        @plsc.parallel_loop(0, GROUPS, unroll=8)
        def group(t):
            rows = idx_v[pl.ds(t * L, L)] * PAD_D
            orow = t * L + iota
            for c in range(OUT_D):
                vals = plsc.load_gather(table_v, [rows + c])
                plsc.store_scatter(out_v, [orow, col_sel[c]], vals)
        pltpu.sync_copy(out_v, out_hbm.at[pl.ds(off, CHUNK)])
        return carry

    lax.fori_loop(0, N_CHUNKS, chunk, 0)


EXP_R = 8192


def _expand_body(in_ref, out_ref):
    out_ref[...] = in_ref[...].reshape(EXP_R, OUT_D)


def kernel(batch, emb_table, W, b):
    wp, bp = _pad_wb(W, b)
    fused = pl.pallas_call(
        _fuse_table_body,
        out_shape=jax.ShapeDtypeStruct((VOCAB, PAD_D), jnp.float32),
    )(emb_table, wp, bp).reshape(VOCAB * PAD_D)

    idx = batch.reshape(TOTAL)

    mesh = plsc.VectorSubcoreMesh(core_axis_name="c", subcore_axis_name="s")
    flat = pl.kernel(
        _gather_body,
        out_type=jax.ShapeDtypeStruct((TOTAL, OUT_D), jnp.float32),
        mesh=mesh,
        scratch_types=[
            pltpu.VMEM((VOCAB * PAD_D,), jnp.float32),
            pltpu.VMEM((CHUNK,), jnp.int32),
            pltpu.VMEM((CHUNK, OUT_D), jnp.float32),
            pltpu.SemaphoreType.DMA,
        ],
        compiler_params=pltpu.CompilerParams(
            use_tc_tiling_on_sc=False, needs_layout_passes=False
        ),
    )(fused, idx)

    return flat.reshape(BATCH, SEQ, OUT_D)


# speedup vs baseline: 1.4120x; 1.0719x over previous
import functools

import jax
import jax.numpy as jnp
from jax import lax
from jax.experimental import pallas as pl
from jax.experimental.pallas import tpu as pltpu
from jax.experimental.pallas import tpu_sc as plsc

NC, NS = 2, 16
NW = NC * NS
OUT_D = 10
PAD_D = 17
L = 16

VOCAB = 1000
BATCH, SEQ = 4096, 200
TOTAL = BATCH * SEQ
N_PER_W = TOTAL // NW
CHUNK = 640
N_CHUNKS = N_PER_W // CHUNK
GROUPS = CHUNK // L


def _fuse_table_body(emb_ref, w_ref, b_ref, out_ref):
    out_ref[...] = (
        jnp.dot(emb_ref[...], w_ref[...], preferred_element_type=jnp.float32)
        + b_ref[...]
    )


def _pad_wb(W, b):
    wp = jnp.zeros((W.shape[0], PAD_D), jnp.float32).at[:, :OUT_D].set(W)
    bp = jnp.zeros((1, PAD_D), jnp.float32).at[0, :OUT_D].set(b)
    return wp, bp


def _gather_body(fused_hbm, idx_hbm, out_hbm, table_v, idx_v, out_v, sem):
    wid = lax.axis_index("s") * NC + lax.axis_index("c")
    base = wid * N_PER_W

    pltpu.sync_copy(fused_hbm, table_v)

    iota = lax.iota(jnp.int32, L)
    col_sel = [jnp.full((L,), c, jnp.int32) for c in range(OUT_D)]

    def chunk(g, carry):
        off = base + g * CHUNK
        pltpu.sync_copy(idx_hbm.at[pl.ds(off, CHUNK)], idx_v)

        @plsc.parallel_loop(0, GROUPS, unroll=8)
        def group(t):
            rows = idx_v[pl.ds(t * L, L)] * PAD_D
            orow = t * L + iota
            for c in range(OUT_D):
                vals = plsc.load_gather(table_v, [rows + c])
                plsc.store_scatter(out_v, [orow, col_sel[c]], vals)
        pltpu.sync_copy(out_v, out_hbm.at[pl.ds(off, CHUNK)])
        return carry

    lax.fori_loop(0, N_CHUNKS, chunk, 0)


EXP_R = 8192


def _expand_body(in_ref, out_ref):
    out_ref[...] = in_ref[...].reshape(EXP_R, OUT_D)


def kernel(batch, emb_table, W, b):
    wp, bp = _pad_wb(W, b)
    fused = pl.pallas_call(
        _fuse_table_body,
        out_shape=jax.ShapeDtypeStruct((VOCAB, PAD_D), jnp.float32),
    )(emb_table, wp, bp).reshape(VOCAB * PAD_D)

    idx = batch.reshape(TOTAL)

    mesh = plsc.VectorSubcoreMesh(core_axis_name="c", subcore_axis_name="s")
    flat = pl.kernel(
        _gather_body,
        out_type=jax.ShapeDtypeStruct((TOTAL, OUT_D), jnp.float32),
        mesh=mesh,
        scratch_types=[
            pltpu.VMEM((VOCAB * PAD_D,), jnp.float32),
            pltpu.VMEM((CHUNK,), jnp.int32),
            pltpu.VMEM((CHUNK, OUT_D), jnp.float32),
            pltpu.SemaphoreType.DMA,
        ],
        compiler_params=pltpu.CompilerParams(
            use_tc_tiling_on_sc=True, needs_layout_passes=False
        ),
    )(fused, idx)

    return flat.reshape(BATCH, SEQ, OUT_D)
